# Initial kernel scaffold; baseline (speedup 1.0000x reference)
import jax
import jax.numpy as jnp
from jax.experimental import pallas as pl


def kernel(idx, vals, weight):
    raise NotImplementedError("write your pallas kernel here")


# speedup vs baseline: 19.9420x; 19.9420x over previous
import functools

import jax
import jax.numpy as jnp
from jax import lax
from jax.experimental import pallas as pl
from jax.experimental.pallas import tpu as pltpu
from jax.experimental.pallas import tpu_sc as plsc

B = 16384
K = 26
VOCAB = 100
DIM = 128

NC = 2
NS = 16
NW = NC * NS
RPW = B // NW
LANES = 16
GROUPS = RPW // LANES

_mesh = plsc.VectorSubcoreMesh(
    core_axis_name="c", subcore_axis_name="s", num_cores=NC, num_subcores=NS
)


@functools.partial(
    pl.kernel,
    out_type=jax.ShapeDtypeStruct((B * VOCAB,), jnp.float32),
    mesh=_mesh,
    scratch_types=[
        pltpu.VMEM((RPW * K,), jnp.int32),
        pltpu.VMEM((RPW * K,), jnp.float32),
        pltpu.VMEM((RPW * VOCAB,), jnp.float32),
    ],
    compiler_params=pltpu.CompilerParams(
        use_tc_tiling_on_sc=False, needs_layout_passes=False
    ),
)
def _hist_kernel(idx_hbm, vals_hbm, h_hbm, idx_v, vals_v, h_v):
    wid = lax.axis_index("s") * NC + lax.axis_index("c")
    ebase = wid * (RPW * K)
    pltpu.sync_copy(idx_hbm.at[pl.ds(ebase, RPW * K)], idx_v)
    pltpu.sync_copy(vals_hbm.at[pl.ds(ebase, RPW * K)], vals_v)

    zeros16 = jnp.zeros((LANES,), jnp.float32)

    def zero_body(i, carry):
        base = i * 128
        for u in range(8):
            h_v[pl.ds(base + u * LANES, LANES)] = zeros16
        return carry

    lax.fori_loop(0, (RPW * VOCAB) // 128, zero_body, 0)

    lane = lax.iota(jnp.int32, LANES)
    lane_k = lane * K
    lane_v = lane * VOCAB

    def scatter_body(g, carry):
        ebase_g = g * (LANES * K) + lane_k
        hbase_g = g * (LANES * VOCAB) + lane_v
        for k in range(K):
            offs = ebase_g + k
            iv = plsc.load_gather(idx_v, [offs])
            vv = plsc.load_gather(vals_v, [offs])
            plsc.addupdate_scatter(h_v, [hbase_g + iv], vv)
        return carry

    lax.fori_loop(0, GROUPS, scatter_body, 0)

    pltpu.sync_copy(h_v, h_hbm.at[pl.ds(wid * (RPW * VOCAB), RPW * VOCAB)])


_BM = 1024


def _mm_body(h_ref, w_ref, o_ref):
    o_ref[:] = jnp.dot(h_ref[:], w_ref[:], preferred_element_type=jnp.float32)


_matmul = pl.pallas_call(
    _mm_body,
    grid=(B // _BM,),
    in_specs=[
        pl.BlockSpec((_BM, VOCAB), lambda i: (i, 0)),
        pl.BlockSpec((VOCAB, DIM), lambda i: (0, 0)),
    ],
    out_specs=pl.BlockSpec((_BM, DIM), lambda i: (i, 0)),
    out_shape=jax.ShapeDtypeStruct((B, DIM), jnp.float32),
)


def kernel(idx, vals, weight):
    idx_flat = idx.astype(jnp.int32).reshape(-1)
    vals_flat = vals.reshape(-1)
    h = _hist_kernel(idx_flat, vals_flat).reshape(B, VOCAB)
    return _matmul(h, weight)
